# baseline (speedup 1.0000x reference)
import functools

import jax
import jax.numpy as jnp
from jax import lax
from jax.experimental import pallas as pl
from jax.experimental.pallas import tpu as pltpu
from jax.experimental.pallas import tpu_sc as plsc

N = 512
D = 128
WIN = 32
SF = float(D - 1)

NC, NS = 2, 16
NW = NC * NS
PPW = D // NW
SLAB = PPW * D * D
SLABPAD = SLAB + 1280


def _win_bounds(c, s):
    c_idx = c * SF
    cut = 3.0 * s * SF
    lof = jnp.floor(jnp.maximum(c_idx - cut, 0.0))
    hif = jnp.minimum(jnp.floor(jnp.minimum(c_idx + cut, SF) + 1.0), float(D))
    return lof, hif


def _win_table(lof, hif, c, s, ii):
    coords = (lof + ii) * jnp.float32(1.0 / SF)
    g = jnp.exp(-0.5 * (coords - c) ** 2 / (s * s))
    return jnp.where(ii < (hif - lof), g, 0.0)


def _tc_body(params_ref, gxw_ref, gyw_ref, gzw_ref, meta_ref):
    ii = lax.broadcasted_iota(jnp.int32, (N, WIN), 1).astype(jnp.float32)
    cx = params_ref[0, :].reshape(N, 1)
    cy = params_ref[1, :].reshape(N, 1)
    cz = params_ref[2, :].reshape(N, 1)
    sg = params_ref[3, :].reshape(N, 1)
    inten = params_ref[4, :].reshape(N, 1)

    lofx, hifx = _win_bounds(cx, sg)
    lofy, hify = _win_bounds(cy, sg)
    lofz, hifz = _win_bounds(cz, sg)

    gxw_ref[...] = _win_table(lofx, hifx, cx, sg, ii) * inten
    gyw_ref[...] = _win_table(lofy, hify, cy, sg, ii)
    gzw_ref[...] = _win_table(lofz, hifz, cz, sg, ii)

    meta_ref[...] = jnp.concatenate(
        [
            lofx.astype(jnp.int32),
            hifx.astype(jnp.int32),
            lofy.astype(jnp.int32),
            (hify - lofy).astype(jnp.int32),
            lofz.astype(jnp.int32),
            (hifz - lofz).astype(jnp.int32),
            jnp.zeros((N, 2), jnp.int32),
        ],
        axis=1,
    )


def _tc_factors(params):
    return pl.pallas_call(
        _tc_body,
        out_shape=[
            jax.ShapeDtypeStruct((N, WIN), jnp.float32),
            jax.ShapeDtypeStruct((N, WIN), jnp.float32),
            jax.ShapeDtypeStruct((N, WIN), jnp.float32),
            jax.ShapeDtypeStruct((N, 8), jnp.int32),
        ],
    )(params)


def _sc_body(gxw_h, gyw_h, gzw_h, meta_h, out_h,
             gxw, gyw, gzw, meta, slab, sem):
    wid = lax.axis_index("c") * NS + lax.axis_index("s")

    cps = [
        pltpu.async_copy(gxw_h, gxw, sem),
        pltpu.async_copy(gyw_h, gyw, sem),
        pltpu.async_copy(gzw_h, gzw, sem),
        pltpu.async_copy(meta_h, meta.at[pl.ds(0, 8 * N)], sem),
    ]

    zero = jnp.zeros((16,), jnp.float32)

    def _zero_body(i, c):
        base = i * 256
        for u in range(16):
            slab[pl.ds(base + u * 16, 16)] = zero
        return c

    lax.fori_loop(0, SLABPAD // 256, _zero_body, 0, unroll=False)

    for cp in cps:
        cp.wait()

    p_base = wid * PPW
    lane = lax.iota(jnp.int32, 16)

    DD = D * D

    def _gauss(n2, c):
        mv = meta[pl.ds(n2 * 16, 16)]
        for h in range(2):
            _gauss_one(n2 * 2 + h, mv, h * 8)
        return c

    def _gauss_one(n, mv, f):
        lox = mv[f + 0]
        hix = mv[f + 1]

        @pl.when((lox < p_base + PPW) & (hix > p_base))
        def _():
            loy = mv[f + 2]
            wy = mv[f + 3]
            loz = mv[f + 4]
            wz = mv[f + 5]
            off = n * WIN
            fz0 = gzw[pl.ds(off, 16)]
            fz1 = gzw[pl.ds(off + 16, 16)]
            fy0 = gyw[pl.ds(off, 16)]
            fy1 = gyw[pl.ds(off + 16, 16)]
            p_lo = jnp.maximum(lox, p_base)
            p_hi = jnp.minimum(hix, p_base + PPW)
            abase = loy * D + loz

            def _mk_plane(with_z1):
                def _plane(p, c2):
                    fx = plsc.load_gather(
                        gxw, [jnp.full((16,), off, jnp.int32) + (p - lox)])
                    fxz0 = fx * fz0
                    fxz1 = fx * fz1
                    pidx = (p - p_base) * DD + abase + lane
                    pidx1 = pidx + 16

                    def _rows(fyv, r0):
                        for r in range(r0, r0 + 8):
                            fyb = jnp.full((16,), fyv[r % 16], jnp.float32)
                            plsc.addupdate_scatter(slab, [pidx + r * D],
                                                   fyb * fxz0)
                            if with_z1:
                                plsc.addupdate_scatter(slab, [pidx1 + r * D],
                                                       fyb * fxz1)

                    _rows(fy0, 0)

                    @pl.when(wy > 8)
                    def _():
                        _rows(fy0, 8)

                    @pl.when(wy > 16)
                    def _():
                        _rows(fy1, 16)

                    @pl.when(wy > 24)
                    def _():
                        _rows(fy1, 24)

                    return c2

                return _plane

            @pl.when(wz <= 16)
            def _():
                lax.fori_loop(p_lo, p_hi, _mk_plane(False), 0)

            @pl.when(wz > 16)
            def _():
                lax.fori_loop(p_lo, p_hi, _mk_plane(True), 0)

    lax.fori_loop(0, N // 2, _gauss, 0, unroll=False)

    pltpu.sync_copy(slab.at[pl.ds(0, SLAB)], out_h.at[pl.ds(wid * SLAB, SLAB)])


@functools.cache
def _sc_accum():
    mesh = plsc.VectorSubcoreMesh(core_axis_name="c", subcore_axis_name="s")
    return pl.kernel(
        _sc_body,
        mesh=mesh,
        compiler_params=pltpu.CompilerParams(needs_layout_passes=False),
        out_type=jax.ShapeDtypeStruct((D * D * D,), jnp.float32),
        scratch_types=[
            pltpu.VMEM((N * WIN,), jnp.float32),
            pltpu.VMEM((N * WIN,), jnp.float32),
            pltpu.VMEM((N * WIN,), jnp.float32),
            pltpu.VMEM((8 * N + 16,), jnp.int32),
            pltpu.VMEM((SLABPAD,), jnp.float32),
            pltpu.SemaphoreType.DMA,
        ],
    )


def kernel(centers, sigmas, intensities):
    params = jnp.zeros((8, N), jnp.float32)
    params = params.at[0].set(centers[:, 0])
    params = params.at[1].set(centers[:, 1])
    params = params.at[2].set(centers[:, 2])
    params = params.at[3].set(sigmas)
    params = params.at[4].set(intensities)

    gxw, gyw, gzw, meta = _tc_factors(params)
    vol = _sc_accum()(gxw.reshape(-1), gyw.reshape(-1), gzw.reshape(-1),
                      meta.reshape(-1))
    return vol.reshape(D, D, D)


# speedup vs baseline: 1.0342x; 1.0342x over previous
import functools

import jax
import jax.numpy as jnp
from jax import lax
from jax.experimental import pallas as pl
from jax.experimental.pallas import tpu as pltpu
from jax.experimental.pallas import tpu_sc as plsc

N = 512
D = 128
WIN = 32
SF = float(D - 1)

NC, NS = 2, 16
NW = NC * NS
PPW = D // NW
SLAB = PPW * D * D
SLABPAD = SLAB + 1280


def _win_bounds(c, s):
    c_idx = c * SF
    cut = 3.0 * s * SF
    lof = jnp.floor(jnp.maximum(c_idx - cut, 0.0))
    hif = jnp.minimum(jnp.floor(jnp.minimum(c_idx + cut, SF) + 1.0), float(D))
    return lof, hif


def _win_table(lof, hif, c, s, ii):
    coords = (lof + ii) * jnp.float32(1.0 / SF)
    g = jnp.exp(-0.5 * (coords - c) ** 2 / (s * s))
    return jnp.where(ii < (hif - lof), g, 0.0)


def _tc_body(params_ref, gxw_ref, gyw_ref, gzw_ref, meta_ref):
    ii = lax.broadcasted_iota(jnp.int32, (N, WIN), 1).astype(jnp.float32)
    cx = params_ref[0, :].reshape(N, 1)
    cy = params_ref[1, :].reshape(N, 1)
    cz = params_ref[2, :].reshape(N, 1)
    sg = params_ref[3, :].reshape(N, 1)
    inten = params_ref[4, :].reshape(N, 1)

    lofx, hifx = _win_bounds(cx, sg)
    lofy, hify = _win_bounds(cy, sg)
    lofz, hifz = _win_bounds(cz, sg)

    gxw_ref[...] = _win_table(lofx, hifx, cx, sg, ii) * inten
    gyw_ref[...] = _win_table(lofy, hify, cy, sg, ii)
    gzw_ref[...] = _win_table(lofz, hifz, cz, sg, ii)

    meta_ref[...] = jnp.concatenate(
        [
            lofx.astype(jnp.int32),
            hifx.astype(jnp.int32),
            lofy.astype(jnp.int32),
            (hify - lofy).astype(jnp.int32),
            lofz.astype(jnp.int32),
            (hifz - lofz).astype(jnp.int32),
            jnp.zeros((N, 2), jnp.int32),
        ],
        axis=1,
    )


def _tc_factors(params):
    return pl.pallas_call(
        _tc_body,
        out_shape=[
            jax.ShapeDtypeStruct((N, WIN), jnp.float32),
            jax.ShapeDtypeStruct((N, WIN), jnp.float32),
            jax.ShapeDtypeStruct((N, WIN), jnp.float32),
            jax.ShapeDtypeStruct((N, 8), jnp.int32),
        ],
    )(params)


def _sc_body(gxw_h, gyw_h, gzw_h, meta_h, out_h,
             gxw, gyw, gzw, meta, slab, sem):
    wid = lax.axis_index("s") * NC + lax.axis_index("c")

    cps = []
    for k in range(8):
        ck = lax.rem(wid + k, 8)
        o = ck * (N // 8) * WIN
        om = ck * (N // 8) * 8
        cps.append(pltpu.async_copy(
            gxw_h.at[pl.ds(o, (N // 8) * WIN)],
            gxw.at[pl.ds(o, (N // 8) * WIN)], sem))
        cps.append(pltpu.async_copy(
            gyw_h.at[pl.ds(o, (N // 8) * WIN)],
            gyw.at[pl.ds(o, (N // 8) * WIN)], sem))
        cps.append(pltpu.async_copy(
            gzw_h.at[pl.ds(o, (N // 8) * WIN)],
            gzw.at[pl.ds(o, (N // 8) * WIN)], sem))
        cps.append(pltpu.async_copy(
            meta_h.at[pl.ds(om, (N // 8) * 8)],
            meta.at[pl.ds(om, (N // 8) * 8)], sem))

    zero = jnp.zeros((16,), jnp.float32)

    def _zero_body(i, c):
        base = i * 256
        for u in range(16):
            slab[pl.ds(base + u * 16, 16)] = zero
        return c

    lax.fori_loop(0, SLABPAD // 256, _zero_body, 0, unroll=False)

    for cp in cps:
        cp.wait()

    p_base = wid * PPW
    lane = lax.iota(jnp.int32, 16)

    DD = D * D

    def _gauss(n2, c):
        mv = meta[pl.ds(n2 * 16, 16)]
        for h in range(2):
            _gauss_one(n2 * 2 + h, mv, h * 8)
        return c

    def _gauss_one(n, mv, f):
        lox = mv[f + 0]
        hix = mv[f + 1]

        @pl.when((lox < p_base + PPW) & (hix > p_base))
        def _():
            loy = mv[f + 2]
            wy = mv[f + 3]
            loz = mv[f + 4]
            wz = mv[f + 5]
            off = n * WIN
            fz0 = gzw[pl.ds(off, 16)]
            fz1 = gzw[pl.ds(off + 16, 16)]
            fy0 = gyw[pl.ds(off, 16)]
            fy1 = gyw[pl.ds(off + 16, 16)]
            p_lo = jnp.maximum(lox, p_base)
            p_hi = jnp.minimum(hix, p_base + PPW)
            abase = loy * D + loz

            def _mk_plane(with_z1):
                def _plane(p, c2):
                    fx = plsc.load_gather(
                        gxw, [jnp.full((16,), off, jnp.int32) + (p - lox)])
                    fxz0 = fx * fz0
                    fxz1 = fx * fz1
                    pidx = (p - p_base) * DD + abase + lane
                    pidx1 = pidx + 16

                    def _rows(fyv, r0):
                        for r in range(r0, r0 + 8):
                            fyb = jnp.full((16,), fyv[r % 16], jnp.float32)
                            plsc.addupdate_scatter(slab, [pidx + r * D],
                                                   fyb * fxz0)
                            if with_z1:
                                plsc.addupdate_scatter(slab, [pidx1 + r * D],
                                                       fyb * fxz1)

                    _rows(fy0, 0)

                    @pl.when(wy > 8)
                    def _():
                        _rows(fy0, 8)

                    @pl.when(wy > 16)
                    def _():
                        _rows(fy1, 16)

                    @pl.when(wy > 24)
                    def _():
                        _rows(fy1, 24)

                    return c2

                return _plane

            @pl.when(wz <= 16)
            def _():
                lax.fori_loop(p_lo, p_hi, _mk_plane(False), 0)

            @pl.when(wz > 16)
            def _():
                lax.fori_loop(p_lo, p_hi, _mk_plane(True), 0)

    lax.fori_loop(0, N // 2, _gauss, 0, unroll=False)

    pltpu.sync_copy(slab.at[pl.ds(0, SLAB)], out_h.at[pl.ds(wid * SLAB, SLAB)])


@functools.cache
def _sc_accum():
    mesh = plsc.VectorSubcoreMesh(core_axis_name="c", subcore_axis_name="s")
    return pl.kernel(
        _sc_body,
        mesh=mesh,
        compiler_params=pltpu.CompilerParams(needs_layout_passes=False),
        out_type=jax.ShapeDtypeStruct((D * D * D,), jnp.float32),
        scratch_types=[
            pltpu.VMEM((N * WIN,), jnp.float32),
            pltpu.VMEM((N * WIN,), jnp.float32),
            pltpu.VMEM((N * WIN,), jnp.float32),
            pltpu.VMEM((8 * N + 16,), jnp.int32),
            pltpu.VMEM((SLABPAD,), jnp.float32),
            pltpu.SemaphoreType.DMA,
        ],
    )


def kernel(centers, sigmas, intensities):
    params = jnp.zeros((8, N), jnp.float32)
    params = params.at[0].set(centers[:, 0])
    params = params.at[1].set(centers[:, 1])
    params = params.at[2].set(centers[:, 2])
    params = params.at[3].set(sigmas)
    params = params.at[4].set(intensities)

    gxw, gyw, gzw, meta = _tc_factors(params)
    vol = _sc_accum()(gxw.reshape(-1), gyw.reshape(-1), gzw.reshape(-1),
                      meta.reshape(-1))
    return vol.reshape(D, D, D)
